# Initial kernel scaffold; baseline (speedup 1.0000x reference)
import jax
import jax.numpy as jnp
from jax.experimental import pallas as pl


def kernel(x, idx, elem, W1, b1, W2, b2, A1, a1, A2, a2):
    raise NotImplementedError("write your pallas kernel here")


# speedup vs baseline: 3.9417x; 3.9417x over previous
import functools

import jax
import jax.numpy as jnp
from jax import lax
from jax.experimental import pallas as pl
from jax.experimental.pallas import tpu as pltpu
from jax.experimental.pallas import tpu_sc as plsc

N = 10000
D = 128
H = 2
AH = 16
E = 320000
R = 144
CH = 128
NCHUNK = E // CH
ZROWS = 80
NZ = N // ZROWS
BN = 1000

_HIGH = lax.Precision.HIGHEST


def _prep_body(x_ref, W1_ref, b1_ref, W2_ref, b2_ref, A1s_ref, A1d_ref,
               a1_ref, S_ref, T_ref):
    h = pl.program_id(0)
    xb = x_ref[...]
    f = jnp.maximum(jnp.dot(xb, W1_ref[0], precision=_HIGH) + b1_ref[h], 0.0)
    f = jnp.dot(f, W2_ref[0], precision=_HIGH) + b2_ref[h]
    S_ref[0] = jnp.dot(f, A1s_ref[0], precision=_HIGH) + a1_ref[h]
    pd = jnp.dot(f, A1d_ref[0], precision=_HIGH)
    T_ref[0] = jnp.concatenate([f, pd], axis=1)


def _prep(x, W1, b1, W2, b2, A1s, A1d, a1):
    grid = (H, N // BN)
    return pl.pallas_call(
        _prep_body,
        grid=grid,
        in_specs=[
            pl.BlockSpec((BN, D), lambda h, i: (i, 0)),
            pl.BlockSpec((1, D, D), lambda h, i: (h, 0, 0)),
            pl.BlockSpec((H, D), lambda h, i: (0, 0)),
            pl.BlockSpec((1, D, D), lambda h, i: (h, 0, 0)),
            pl.BlockSpec((H, D), lambda h, i: (0, 0)),
            pl.BlockSpec((1, D, AH), lambda h, i: (h, 0, 0)),
            pl.BlockSpec((1, D, AH), lambda h, i: (h, 0, 0)),
            pl.BlockSpec((H, AH), lambda h, i: (0, 0)),
        ],
        out_specs=[
            pl.BlockSpec((1, BN, AH), lambda h, i: (h, i, 0)),
            pl.BlockSpec((1, BN, R), lambda h, i: (h, i, 0)),
        ],
        out_shape=[
            jax.ShapeDtypeStruct((H, N, AH), jnp.float32),
            jax.ShapeDtypeStruct((H, N, R), jnp.float32),
        ],
    )(x, W1, b1, W2, b2, A1s, A1d, a1)


def _edge_body(S_hbm, T_hbm, src_hbm, dst_hbm, elem_hbm, consts_hbm,
               out_hbm, acc, cbuf, sidx, sadj, didx, g1v, g2v, elv, scv,
               zbuf, sem0, sem1):
    h = lax.axis_index("c")
    tid = lax.axis_index("s")
    hN = h * N

    pltpu.sync_copy(consts_hbm.at[h], cbuf)
    c_vec = cbuf[0]
    a2v = cbuf[1]
    a2rep = cbuf[2]

    def _zrow(i, _):
        for k in range(R // 16):
            zbuf[i, pl.ds(k * 16, 16)] = jnp.zeros((16,), jnp.float32)
        return 0

    lax.fori_loop(0, ZROWS, _zrow, 0)

    def _zchunk(j, _):
        z = j * 16 + tid

        @pl.when(z < NZ)
        def _():
            pltpu.sync_copy(zbuf, acc.at[pl.ds(z * ZROWS, ZROWS)])

        return 0

    lax.fori_loop(0, (NZ + 15) // 16, _zchunk, 0)
    plsc.subcore_barrier()

    def _chunk(j, _):
        cid = j * 16 + tid

        @pl.when(cid < NCHUNK)
        def _():
            base = cid * CH
            pltpu.sync_copy(src_hbm.at[pl.ds(base, CH)], sidx)
            pltpu.sync_copy(dst_hbm.at[pl.ds(base, CH)], didx)
            pltpu.sync_copy(elem_hbm.at[pl.ds(base, CH)], elv.at[pl.ds(0, CH)])
            for k in range(CH // 16):
                sl = pl.ds(k * 16, 16)
                sadj[sl] = sidx[sl] + hN
                didx[sl] = didx[sl] + hN
            pltpu.async_copy(S_hbm.at[sadj], g1v, sem0).wait()
            pltpu.async_copy(T_hbm.at[didx], g2v, sem1).wait()

            lane = lax.iota(jnp.int32, 16)

            def _group(g, _):
                g16 = g * 16
                el = elv[pl.ds(g16, 16)]
                row = lane + g16
                sc = a2rep
                for j in range(AH):
                    cj = jnp.full((16,), j, jnp.int32)
                    u = (plsc.load_gather(g1v, [row, cj])
                         + plsc.load_gather(g2v, [row, cj + D])
                         + el * c_vec[j])
                    sc = sc + jnp.maximum(u, 0.0) * a2v[j]
                ev = jnp.exp(jnp.where(sc > 0, sc, 0.2 * sc))
                scv[pl.ds(g16, 16)] = ev
                return 0

            lax.fori_loop(0, CH // 16, _group, 0)

            def _scale(i, _):
                e = scv[pl.ds(i, 16)][0]
                for k in range(D // 16):
                    sl = pl.ds(k * 16, 16)
                    g2v[i, sl] = g2v[i, sl] * e
                g2v[i, pl.ds(D, 16)] = jnp.where(lane == 0, e, 0.0)
                return 0

            lax.fori_loop(0, CH, _scale, 0)
            pltpu.sync_copy(g2v, acc.at[sidx], add=True)

        return 0

    lax.fori_loop(0, (NCHUNK + 15) // 16, _chunk, 0)
    plsc.subcore_barrier()

    def _dump(j, _):
        z = j * 16 + tid

        @pl.when(z < NZ)
        def _():
            pltpu.sync_copy(acc.at[pl.ds(z * ZROWS, ZROWS)], zbuf)
            pltpu.sync_copy(zbuf, out_hbm.at[pl.ds(hN + z * ZROWS, ZROWS)])

        return 0

    lax.fori_loop(0, (NZ + 15) // 16, _dump, 0)


def _edge_pass(S2, T2, src, dst, elem, consts):
    mesh = plsc.VectorSubcoreMesh(core_axis_name="c", subcore_axis_name="s")
    k = pl.kernel(
        _edge_body,
        out_type=jax.ShapeDtypeStruct((H * N, R), jnp.float32),
        mesh=mesh,
        compiler_params=pltpu.CompilerParams(
            needs_layout_passes=False, use_tc_tiling_on_sc=False),
        scratch_types=[
            pltpu.VMEM_SHARED((N, R), jnp.float32),
            pltpu.VMEM((4, 16), jnp.float32),
            pltpu.VMEM((CH,), jnp.int32),
            pltpu.VMEM((CH,), jnp.int32),
            pltpu.VMEM((CH,), jnp.int32),
            pltpu.VMEM((CH, AH), jnp.float32),
            pltpu.VMEM((CH, R), jnp.float32),
            pltpu.VMEM((CH + 16,), jnp.float32),
            pltpu.VMEM((CH + 16,), jnp.float32),
            pltpu.VMEM((ZROWS, R), jnp.float32),
            pltpu.SemaphoreType.DMA,
            pltpu.SemaphoreType.DMA,
        ],
    )
    return k(S2, T2, src, dst, elem, consts)


def _fin_body(P_ref, o_ref):
    pb = P_ref[0]
    o_ref[...] = pb[:, :D] / (pb[:, D:D + 1] + 1e-10)


def _finalize(P):
    grid = (H, N // BN)
    return pl.pallas_call(
        _fin_body,
        grid=grid,
        in_specs=[pl.BlockSpec((1, BN, R), lambda h, i: (h, i, 0))],
        out_specs=pl.BlockSpec((BN, D), lambda h, i: (i, h)),
        out_shape=jax.ShapeDtypeStruct((N, H * D), jnp.float32),
    )(P)


def kernel(x, idx, elem, W1, b1, W2, b2, A1, a1, A2, a2):
    A1s = A1[:, :D, :]
    A1d = A1[:, D:2 * D, :]
    c = A1[:, 2 * D, :]
    a2v = A2[:, :, 0]
    a2rep = jnp.broadcast_to(a2, (H, 16))
    consts = jnp.stack([c, a2v, a2rep, jnp.zeros_like(c)], axis=1)

    S, T = _prep(x, W1, b1, W2, b2, A1s, A1d, a1)
    S2 = S.reshape(H * N, AH)
    T2 = T.reshape(H * N, R)
    src = idx[0]
    dst = idx[1]
    P = _edge_pass(S2, T2, src, dst, elem, consts)
    return _finalize(P.reshape(H, N, R))


# speedup vs baseline: 5.3254x; 1.3510x over previous
import functools

import jax
import jax.numpy as jnp
from jax import lax
from jax.experimental import pallas as pl
from jax.experimental.pallas import tpu as pltpu
from jax.experimental.pallas import tpu_sc as plsc

N = 10000
D = 128
H = 2
AH = 16
E = 320000
R = 144
CH = 64
NCHUNK = E // CH
NZC = N // CH
ZTAIL = N - NZC * CH
BN = 1000

_HIGH = lax.Precision.HIGHEST


def _prep_body(x_ref, W1_ref, b1_ref, W2_ref, b2_ref, A1s_ref, A1d_ref,
               a1_ref, S_ref, T_ref):
    h = pl.program_id(0)
    xb = x_ref[...]
    f = jnp.maximum(jnp.dot(xb, W1_ref[0], precision=_HIGH) + b1_ref[h], 0.0)
    f = jnp.dot(f, W2_ref[0], precision=_HIGH) + b2_ref[h]
    S_ref[0] = jnp.dot(f, A1s_ref[0], precision=_HIGH) + a1_ref[h]
    pd = jnp.dot(f, A1d_ref[0], precision=_HIGH)
    T_ref[0] = jnp.concatenate([f, pd], axis=1)


def _prep(x, W1, b1, W2, b2, A1s, A1d, a1):
    grid = (H, N // BN)
    return pl.pallas_call(
        _prep_body,
        grid=grid,
        in_specs=[
            pl.BlockSpec((BN, D), lambda h, i: (i, 0)),
            pl.BlockSpec((1, D, D), lambda h, i: (h, 0, 0)),
            pl.BlockSpec((H, D), lambda h, i: (0, 0)),
            pl.BlockSpec((1, D, D), lambda h, i: (h, 0, 0)),
            pl.BlockSpec((H, D), lambda h, i: (0, 0)),
            pl.BlockSpec((1, D, AH), lambda h, i: (h, 0, 0)),
            pl.BlockSpec((1, D, AH), lambda h, i: (h, 0, 0)),
            pl.BlockSpec((H, AH), lambda h, i: (0, 0)),
        ],
        out_specs=[
            pl.BlockSpec((1, BN, AH), lambda h, i: (h, i, 0)),
            pl.BlockSpec((1, BN, R), lambda h, i: (h, i, 0)),
        ],
        out_shape=[
            jax.ShapeDtypeStruct((H, N, AH), jnp.float32),
            jax.ShapeDtypeStruct((H, N, R), jnp.float32),
        ],
    )(x, W1, b1, W2, b2, A1s, A1d, a1)


NSLOT = 3
NITER = (NCHUNK + 15) // 16
NOUTER = (NITER + NSLOT) // NSLOT


def _edge_body(S_hbm, T_hbm, srcr_hbm, sadj_hbm, dadj_hbm, elem_hbm,
               consts_hbm, out_hbm, acc, cbuf, isrc, iga, igb, elv, g1v,
               g2v, scv, semA, semB, semS):
    h = lax.axis_index("c")
    tid = lax.axis_index("s")
    hN = h * N
    hE = h * E

    pltpu.sync_copy(consts_hbm.at[h], cbuf)
    c_vec = cbuf[0]
    a2v = cbuf[1]
    a2rep = cbuf[2]

    def _zrow(i, _):
        for k in range(R // 16):
            g2v[0][i, pl.ds(k * 16, 16)] = jnp.zeros((16,), jnp.float32)
        return 0

    lax.fori_loop(0, CH, _zrow, 0)

    def _zchunk(j, _):
        z = j * 16 + tid

        @pl.when(z < NZC)
        def _():
            pltpu.sync_copy(g2v[0], acc.at[pl.ds(z * CH, CH)])

        @pl.when(z == NZC)
        def _():
            pltpu.sync_copy(g2v[0].at[pl.ds(0, ZTAIL)],
                            acc.at[pl.ds(NZC * CH, ZTAIL)])

        return 0

    lax.fori_loop(0, (NZC + 16) // 16, _zchunk, 0)
    plsc.subcore_barrier()

    def _valid(j):
        return (j * 16 + tid) < NCHUNK

    def _base(j):
        return (j * 16 + tid) * CH

    def _issue_a(j, s):
        b = _base(j)
        pltpu.async_copy(srcr_hbm.at[pl.ds(b, CH)], isrc[s], semA[s])
        pltpu.async_copy(sadj_hbm.at[pl.ds(hE + b, CH)], iga[s], semA[s])
        pltpu.async_copy(dadj_hbm.at[pl.ds(hE + b, CH)], igb[s], semA[s])
        pltpu.async_copy(elem_hbm.at[pl.ds(b, CH)],
                         elv[s].at[pl.ds(0, CH)], semA[s])

    def _wait_a(s):
        pltpu.make_async_copy(srcr_hbm.at[pl.ds(0, CH)], isrc[s], semA[s]).wait()
        pltpu.make_async_copy(sadj_hbm.at[pl.ds(0, CH)], iga[s], semA[s]).wait()
        pltpu.make_async_copy(dadj_hbm.at[pl.ds(0, CH)], igb[s], semA[s]).wait()
        pltpu.make_async_copy(elem_hbm.at[pl.ds(0, CH)],
                              elv[s].at[pl.ds(0, CH)], semA[s]).wait()

    def _issue_b(s):
        pltpu.async_copy(S_hbm.at[iga[s]], g1v[s], semB[s])
        pltpu.async_copy(T_hbm.at[igb[s]], g2v[s], semB[s])

    def _wait_b(s):
        pltpu.make_async_copy(S_hbm.at[pl.ds(0, CH)], g1v[s], semB[s]).wait()
        pltpu.make_async_copy(T_hbm.at[pl.ds(0, CH)], g2v[s], semB[s]).wait()

    def _issue_s(s):
        pltpu.async_copy(g2v[s], acc.at[isrc[s]], semS[s], add=True)

    def _wait_s(s):
        pltpu.make_async_copy(g2v[s], acc.at[pl.ds(0, CH)], semS[s]).wait()

    lane = lax.iota(jnp.int32, 16)

    def _compute(s):
        def _group(g, _):
            g16 = g * 16
            el = elv[s][pl.ds(g16, 16)]
            row = lane + g16
            sc = a2rep
            for jj in range(AH):
                cj = jnp.full((16,), jj, jnp.int32)
                u = (plsc.load_gather(g1v[s], [row, cj])
                     + plsc.load_gather(g2v[s], [row, cj + D])
                     + el * c_vec[jj])
                sc = sc + jnp.maximum(u, 0.0) * a2v[jj]
            ev = jnp.exp(jnp.where(sc > 0, sc, 0.2 * sc))
            scv[pl.ds(g16, 16)] = ev
            return 0

        lax.fori_loop(0, CH // 16, _group, 0)

        def _scale(i, _):
            e = scv[pl.ds(i, 16)][0]
            for k in range(D // 16):
                sl = pl.ds(k * 16, 16)
                g2v[s][i, sl] = g2v[s][i, sl] * e
            g2v[s][i, pl.ds(D, 16)] = jnp.where(lane == 0, e, 0.0)
            return 0

        lax.fori_loop(0, CH, _scale, 0)

    _issue_a(0, 0)
    _issue_a(1, 1)
    _wait_a(0)
    _issue_b(0)

    def _iter(j, s, s1, s2):
        @pl.when(_valid(j))
        def _():
            _wait_b(s)
            _compute(s)
            _issue_s(s)

        @pl.when((j >= 1) & _valid(j - 1))
        def _():
            _wait_s(s2)

        @pl.when(_valid(j + 2))
        def _():
            _issue_a(j + 2, s2)

        @pl.when(_valid(j + 1))
        def _():
            _wait_a(s1)
            _issue_b(s1)

    def _outer(jj, _):
        j0 = jj * NSLOT
        for b in range(NSLOT):
            _iter(j0 + b, b % NSLOT, (b + 1) % NSLOT, (b + 2) % NSLOT)
        return 0

    lax.fori_loop(0, NOUTER, _outer, 0)
    plsc.subcore_barrier()

    def _dump(j, _):
        z = j * 16 + tid

        @pl.when(z < NZC)
        def _():
            pltpu.sync_copy(acc.at[pl.ds(z * CH, CH)], g2v[0])
            pltpu.sync_copy(g2v[0], out_hbm.at[pl.ds(hN + z * CH, CH)])

        @pl.when(z == NZC)
        def _():
            pltpu.sync_copy(acc.at[pl.ds(NZC * CH, ZTAIL)],
                            g2v[0].at[pl.ds(0, ZTAIL)])
            pltpu.sync_copy(g2v[0].at[pl.ds(0, ZTAIL)],
                            out_hbm.at[pl.ds(hN + NZC * CH, ZTAIL)])

        return 0

    lax.fori_loop(0, (NZC + 16) // 16, _dump, 0)


def _edge_pass(S2, T2, srcr, sadj2, dadj2, elem, consts):
    mesh = plsc.VectorSubcoreMesh(core_axis_name="c", subcore_axis_name="s")
    nsl = NSLOT
    k = pl.kernel(
        _edge_body,
        out_type=jax.ShapeDtypeStruct((H * N, R), jnp.float32),
        mesh=mesh,
        compiler_params=pltpu.CompilerParams(
            needs_layout_passes=False, use_tc_tiling_on_sc=False),
        scratch_types=[
            pltpu.VMEM_SHARED((N, R), jnp.float32),
            pltpu.VMEM((4, 16), jnp.float32),
            tuple(pltpu.VMEM((CH,), jnp.int32) for _ in range(nsl)),
            tuple(pltpu.VMEM((CH,), jnp.int32) for _ in range(nsl)),
            tuple(pltpu.VMEM((CH,), jnp.int32) for _ in range(nsl)),
            tuple(pltpu.VMEM((CH + 16,), jnp.float32) for _ in range(nsl)),
            tuple(pltpu.VMEM((CH, AH), jnp.float32) for _ in range(nsl)),
            tuple(pltpu.VMEM((CH, R), jnp.float32) for _ in range(nsl)),
            pltpu.VMEM((CH + 16,), jnp.float32),
            tuple(pltpu.SemaphoreType.DMA for _ in range(nsl)),
            tuple(pltpu.SemaphoreType.DMA for _ in range(nsl)),
            tuple(pltpu.SemaphoreType.DMA for _ in range(nsl)),
        ],
    )
    return k(S2, T2, srcr, sadj2, dadj2, elem, consts)


def _fin_body(P_ref, o_ref):
    pb = P_ref[0]
    o_ref[...] = pb[:, :D] / (pb[:, D:D + 1] + 1e-10)


def _finalize(P):
    grid = (H, N // BN)
    return pl.pallas_call(
        _fin_body,
        grid=grid,
        in_specs=[pl.BlockSpec((1, BN, R), lambda h, i: (h, i, 0))],
        out_specs=pl.BlockSpec((BN, D), lambda h, i: (i, h)),
        out_shape=jax.ShapeDtypeStruct((N, H * D), jnp.float32),
    )(P)


def kernel(x, idx, elem, W1, b1, W2, b2, A1, a1, A2, a2):
    A1s = A1[:, :D, :]
    A1d = A1[:, D:2 * D, :]
    c = A1[:, 2 * D, :]
    a2v = A2[:, :, 0]
    a2rep = jnp.broadcast_to(a2, (H, 16))
    consts = jnp.stack([c, a2v, a2rep, jnp.zeros_like(c)], axis=1)

    S, T = _prep(x, W1, b1, W2, b2, A1s, A1d, a1)
    S2 = S.reshape(H * N, AH)
    T2 = T.reshape(H * N, R)
    src = idx[0]
    dst = idx[1]
    sadj2 = jnp.concatenate([src, src + N]).astype(jnp.int32)
    dadj2 = jnp.concatenate([dst, dst + N]).astype(jnp.int32)
    P = _edge_pass(S2, T2, src, sadj2, dadj2, elem, consts)
    return _finalize(P.reshape(H, N, R))


# speedup vs baseline: 6.8175x; 1.2802x over previous
import functools

import jax
import jax.numpy as jnp
from jax import lax
from jax.experimental import pallas as pl
from jax.experimental.pallas import tpu as pltpu
from jax.experimental.pallas import tpu_sc as plsc

N = 10000
D = 128
H = 2
AH = 16
E = 320000
R = 144
CH = 64
NCHUNK = E // CH
NZC = N // CH
ZTAIL = N - NZC * CH
BN = 1000

_HIGH = lax.Precision.HIGHEST


def _prep_body(x_ref, W1_ref, b1_ref, W2_ref, b2_ref, A1s_ref, A1d_ref,
               a1_ref, S_ref, T_ref):
    h = pl.program_id(0)
    xb = x_ref[...]
    f = jnp.maximum(jnp.dot(xb, W1_ref[0], precision=_HIGH) + b1_ref[h], 0.0)
    f = jnp.dot(f, W2_ref[0], precision=_HIGH) + b2_ref[h]
    S_ref[0] = jnp.dot(f, A1s_ref[0], precision=_HIGH) + a1_ref[h]
    pd = jnp.dot(f, A1d_ref[0], precision=_HIGH)
    T_ref[0] = jnp.concatenate([f, pd], axis=1)


def _prep(x, W1, b1, W2, b2, A1s, A1d, a1):
    grid = (H, N // BN)
    return pl.pallas_call(
        _prep_body,
        grid=grid,
        in_specs=[
            pl.BlockSpec((BN, D), lambda h, i: (i, 0)),
            pl.BlockSpec((1, D, D), lambda h, i: (h, 0, 0)),
            pl.BlockSpec((H, D), lambda h, i: (0, 0)),
            pl.BlockSpec((1, D, D), lambda h, i: (h, 0, 0)),
            pl.BlockSpec((H, D), lambda h, i: (0, 0)),
            pl.BlockSpec((1, D, AH), lambda h, i: (h, 0, 0)),
            pl.BlockSpec((1, D, AH), lambda h, i: (h, 0, 0)),
            pl.BlockSpec((H, AH), lambda h, i: (0, 0)),
        ],
        out_specs=[
            pl.BlockSpec((1, BN, AH), lambda h, i: (h, i, 0)),
            pl.BlockSpec((1, BN, R), lambda h, i: (h, i, 0)),
        ],
        out_shape=[
            jax.ShapeDtypeStruct((H, N, AH), jnp.float32),
            jax.ShapeDtypeStruct((H, N, R), jnp.float32),
        ],
    )(x, W1, b1, W2, b2, A1s, A1d, a1)


NSLOT = 3
NITER = (NCHUNK + 15) // 16
NOUTER = (NITER + NSLOT) // NSLOT


def _edge_body(S_hbm, T_hbm, srcr_hbm, sadj_hbm, dadj_hbm, elem_hbm,
               consts_hbm, out_hbm, acc, cbuf, isrc, iga, igb, elv, g1v,
               g2v, scv, semA, semB, semS):
    h = lax.axis_index("c")
    tid = lax.axis_index("s")
    hN = h * N
    hE = h * E

    pltpu.sync_copy(consts_hbm.at[h], cbuf)
    c_vec = cbuf[0]
    a2v = cbuf[1]
    a2rep = cbuf[2]

    def _zrow(i, _):
        for k in range(R // 16):
            g2v[0][i, pl.ds(k * 16, 16)] = jnp.zeros((16,), jnp.float32)
        return 0

    lax.fori_loop(0, CH, _zrow, 0)

    def _zchunk(j, _):
        z = j * 16 + tid

        @pl.when(z < NZC)
        def _():
            pltpu.sync_copy(g2v[0], acc.at[pl.ds(z * CH, CH)])

        @pl.when(z == NZC)
        def _():
            pltpu.sync_copy(g2v[0].at[pl.ds(0, ZTAIL)],
                            acc.at[pl.ds(NZC * CH, ZTAIL)])

        return 0

    lax.fori_loop(0, (NZC + 16) // 16, _zchunk, 0)
    plsc.subcore_barrier()

    def _valid(j):
        return (j * 16 + tid) < NCHUNK

    def _base(j):
        return (j * 16 + tid) * CH

    def _issue_a(j, s):
        b = _base(j)
        pltpu.async_copy(srcr_hbm.at[pl.ds(b, CH)], isrc[s], semA[s])
        pltpu.async_copy(sadj_hbm.at[pl.ds(hE + b, CH)], iga[s], semA[s])
        pltpu.async_copy(dadj_hbm.at[pl.ds(hE + b, CH)], igb[s], semA[s])
        pltpu.async_copy(elem_hbm.at[pl.ds(b, CH)],
                         elv[s].at[pl.ds(0, CH)], semA[s])

    def _wait_a(s):
        pltpu.make_async_copy(srcr_hbm.at[pl.ds(0, CH)], isrc[s], semA[s]).wait()
        pltpu.make_async_copy(sadj_hbm.at[pl.ds(0, CH)], iga[s], semA[s]).wait()
        pltpu.make_async_copy(dadj_hbm.at[pl.ds(0, CH)], igb[s], semA[s]).wait()
        pltpu.make_async_copy(elem_hbm.at[pl.ds(0, CH)],
                              elv[s].at[pl.ds(0, CH)], semA[s]).wait()

    def _issue_b(s):
        pltpu.async_copy(S_hbm.at[iga[s]], g1v[s], semB[s])
        pltpu.async_copy(T_hbm.at[igb[s]], g2v[s], semB[s])

    def _wait_b(s):
        pltpu.make_async_copy(S_hbm.at[pl.ds(0, CH)], g1v[s], semB[s]).wait()
        pltpu.make_async_copy(T_hbm.at[pl.ds(0, CH)], g2v[s], semB[s]).wait()

    def _issue_s(s):
        pltpu.async_copy(g2v[s], acc.at[isrc[s]], semS[s], add=True)

    def _wait_s(s):
        pltpu.make_async_copy(g2v[s], acc.at[pl.ds(0, CH)], semS[s]).wait()

    lane = lax.iota(jnp.int32, 16)

    def _compute(s):
        def _group(g, _):
            g16 = g * 16
            el = elv[s][pl.ds(g16, 16)]
            row = lane + g16
            sc = a2rep
            for jj in range(AH):
                cj = jnp.full((16,), jj, jnp.int32)
                u = (plsc.load_gather(g1v[s], [row, cj])
                     + plsc.load_gather(g2v[s], [row, cj + D])
                     + el * c_vec[jj])
                sc = sc + jnp.maximum(u, 0.0) * a2v[jj]
            ev = jnp.exp(jnp.where(sc > 0, sc, 0.2 * sc))
            scv[pl.ds(g16, 16)] = ev
            return 0

        lax.fori_loop(0, CH // 16, _group, 0)

        def _scale(i, _):
            e = scv[pl.ds(i, 16)][0]
            for k in range(D // 16):
                sl = pl.ds(k * 16, 16)
                g2v[s][i, sl] = g2v[s][i, sl] * e
            g2v[s][i, pl.ds(D, 16)] = jnp.where(lane == 0, e, 0.0)
            return 0

        lax.fori_loop(0, CH, _scale, 0)

    _issue_a(0, 0)
    _issue_a(1, 1)
    _wait_a(0)
    _issue_b(0)

    def _iter(j, s, s1, s2):
        @pl.when(_valid(j))
        def _():
            _wait_b(s)

        @pl.when(_valid(j + 1))
        def _():
            _wait_a(s1)
            _issue_b(s1)

        @pl.when(_valid(j))
        def _():
            _compute(s)
            _issue_s(s)

        @pl.when((j >= 1) & _valid(j - 1))
        def _():
            _wait_s(s2)

        @pl.when(_valid(j + 2))
        def _():
            _issue_a(j + 2, s2)

    def _outer(jj, _):
        j0 = jj * NSLOT
        for b in range(NSLOT):
            _iter(j0 + b, b % NSLOT, (b + 1) % NSLOT, (b + 2) % NSLOT)
        return 0

    lax.fori_loop(0, NOUTER, _outer, 0)
    plsc.subcore_barrier()

    def _dump(j, _):
        z = j * 16 + tid

        @pl.when(z < NZC)
        def _():
            pltpu.sync_copy(acc.at[pl.ds(z * CH, CH)], g2v[0])
            pltpu.sync_copy(g2v[0], out_hbm.at[pl.ds(hN + z * CH, CH)])

        @pl.when(z == NZC)
        def _():
            pltpu.sync_copy(acc.at[pl.ds(NZC * CH, ZTAIL)],
                            g2v[0].at[pl.ds(0, ZTAIL)])
            pltpu.sync_copy(g2v[0].at[pl.ds(0, ZTAIL)],
                            out_hbm.at[pl.ds(hN + NZC * CH, ZTAIL)])

        return 0

    lax.fori_loop(0, (NZC + 16) // 16, _dump, 0)


def _edge_pass(S2, T2, srcr, sadj2, dadj2, elem, consts):
    mesh = plsc.VectorSubcoreMesh(core_axis_name="c", subcore_axis_name="s")
    nsl = NSLOT
    k = pl.kernel(
        _edge_body,
        out_type=jax.ShapeDtypeStruct((H * N, R), jnp.float32),
        mesh=mesh,
        compiler_params=pltpu.CompilerParams(
            needs_layout_passes=False, use_tc_tiling_on_sc=False),
        scratch_types=[
            pltpu.VMEM_SHARED((N, R), jnp.float32),
            pltpu.VMEM((4, 16), jnp.float32),
            tuple(pltpu.VMEM((CH,), jnp.int32) for _ in range(nsl)),
            tuple(pltpu.VMEM((CH,), jnp.int32) for _ in range(nsl)),
            tuple(pltpu.VMEM((CH,), jnp.int32) for _ in range(nsl)),
            tuple(pltpu.VMEM((CH + 16,), jnp.float32) for _ in range(nsl)),
            tuple(pltpu.VMEM((CH, AH), jnp.float32) for _ in range(nsl)),
            tuple(pltpu.VMEM((CH, R), jnp.float32) for _ in range(nsl)),
            pltpu.VMEM((CH + 16,), jnp.float32),
            tuple(pltpu.SemaphoreType.DMA for _ in range(nsl)),
            tuple(pltpu.SemaphoreType.DMA for _ in range(nsl)),
            tuple(pltpu.SemaphoreType.DMA for _ in range(nsl)),
        ],
    )
    return k(S2, T2, srcr, sadj2, dadj2, elem, consts)


def _fin_body(P_ref, o_ref):
    pb = P_ref[0]
    o_ref[...] = pb[:, :D] / (pb[:, D:D + 1] + 1e-10)


def _finalize(P):
    grid = (H, N // BN)
    return pl.pallas_call(
        _fin_body,
        grid=grid,
        in_specs=[pl.BlockSpec((1, BN, R), lambda h, i: (h, i, 0))],
        out_specs=pl.BlockSpec((BN, D), lambda h, i: (i, h)),
        out_shape=jax.ShapeDtypeStruct((N, H * D), jnp.float32),
    )(P)


def kernel(x, idx, elem, W1, b1, W2, b2, A1, a1, A2, a2):
    A1s = A1[:, :D, :]
    A1d = A1[:, D:2 * D, :]
    c = A1[:, 2 * D, :]
    a2v = A2[:, :, 0]
    a2rep = jnp.broadcast_to(a2, (H, 16))
    consts = jnp.stack([c, a2v, a2rep, jnp.zeros_like(c)], axis=1)

    S, T = _prep(x, W1, b1, W2, b2, A1s, A1d, a1)
    S2 = S.reshape(H * N, AH)
    T2 = T.reshape(H * N, R)
    src = idx[0]
    dst = idx[1]
    sadj2 = jnp.concatenate([src, src + N]).astype(jnp.int32)
    dadj2 = jnp.concatenate([dst, dst + N]).astype(jnp.int32)
    P = _edge_pass(S2, T2, src, sadj2, dadj2, elem, consts)
    return _finalize(P.reshape(H, N, R))


# speedup vs baseline: 7.0908x; 1.0401x over previous
import functools

import jax
import jax.numpy as jnp
from jax import lax
from jax.experimental import pallas as pl
from jax.experimental.pallas import tpu as pltpu
from jax.experimental.pallas import tpu_sc as plsc

N = 10000
D = 128
H = 2
AH = 16
E = 320000
R = 144
CH = 80
NCHUNK = E // CH
NZC = N // CH
ZTAIL = N - NZC * CH
BN = 1000

_HIGH = lax.Precision.HIGHEST


def _prep_body(x_ref, W1_ref, b1_ref, W2_ref, b2_ref, A1s_ref, A1d_ref,
               a1_ref, S_ref, T_ref):
    h = pl.program_id(0)
    xb = x_ref[...]
    f = jnp.maximum(jnp.dot(xb, W1_ref[0], precision=_HIGH) + b1_ref[h], 0.0)
    f = jnp.dot(f, W2_ref[0], precision=_HIGH) + b2_ref[h]
    S_ref[0] = jnp.dot(f, A1s_ref[0], precision=_HIGH) + a1_ref[h]
    pd = jnp.dot(f, A1d_ref[0], precision=_HIGH)
    T_ref[0] = jnp.concatenate([f, pd], axis=1)


def _prep(x, W1, b1, W2, b2, A1s, A1d, a1):
    grid = (H, N // BN)
    return pl.pallas_call(
        _prep_body,
        grid=grid,
        in_specs=[
            pl.BlockSpec((BN, D), lambda h, i: (i, 0)),
            pl.BlockSpec((1, D, D), lambda h, i: (h, 0, 0)),
            pl.BlockSpec((H, D), lambda h, i: (0, 0)),
            pl.BlockSpec((1, D, D), lambda h, i: (h, 0, 0)),
            pl.BlockSpec((H, D), lambda h, i: (0, 0)),
            pl.BlockSpec((1, D, AH), lambda h, i: (h, 0, 0)),
            pl.BlockSpec((1, D, AH), lambda h, i: (h, 0, 0)),
            pl.BlockSpec((H, AH), lambda h, i: (0, 0)),
        ],
        out_specs=[
            pl.BlockSpec((1, BN, AH), lambda h, i: (h, i, 0)),
            pl.BlockSpec((1, BN, R), lambda h, i: (h, i, 0)),
        ],
        out_shape=[
            jax.ShapeDtypeStruct((H, N, AH), jnp.float32),
            jax.ShapeDtypeStruct((H, N, R), jnp.float32),
        ],
    )(x, W1, b1, W2, b2, A1s, A1d, a1)


NSLOT = 3
NITER = (NCHUNK + 15) // 16
NOUTER = (NITER + NSLOT) // NSLOT


def _edge_body(S_hbm, T_hbm, srcr_hbm, sadj_hbm, dadj_hbm, elem_hbm,
               consts_hbm, out_hbm, acc, cbuf, isrc, iga, igb, elv, g1v,
               g2v, scv, semA, semB, semS):
    h = lax.axis_index("c")
    tid = lax.axis_index("s")
    hN = h * N
    hE = h * E

    pltpu.sync_copy(consts_hbm.at[h], cbuf)
    c_vec = cbuf[0]
    a2v = cbuf[1]
    a2rep = cbuf[2]

    def _zrow(i, _):
        for k in range(R // 16):
            g2v[0][i, pl.ds(k * 16, 16)] = jnp.zeros((16,), jnp.float32)
        return 0

    lax.fori_loop(0, CH, _zrow, 0)

    def _zchunk(j, _):
        z = j * 16 + tid

        @pl.when(z < NZC)
        def _():
            pltpu.sync_copy(g2v[0], acc.at[pl.ds(z * CH, CH)])

        if ZTAIL:
            @pl.when(z == NZC)
            def _():
                pltpu.sync_copy(g2v[0].at[pl.ds(0, ZTAIL)],
                                acc.at[pl.ds(NZC * CH, ZTAIL)])

        return 0

    lax.fori_loop(0, (NZC + 16) // 16, _zchunk, 0)
    plsc.subcore_barrier()

    def _valid(j):
        return (j * 16 + tid) < NCHUNK

    def _base(j):
        return (j * 16 + tid) * CH

    def _issue_a(j, s):
        b = _base(j)
        pltpu.async_copy(srcr_hbm.at[pl.ds(b, CH)], isrc[s], semA[s])
        pltpu.async_copy(sadj_hbm.at[pl.ds(hE + b, CH)], iga[s], semA[s])
        pltpu.async_copy(dadj_hbm.at[pl.ds(hE + b, CH)], igb[s], semA[s])
        pltpu.async_copy(elem_hbm.at[pl.ds(b, CH)],
                         elv[s].at[pl.ds(0, CH)], semA[s])

    def _wait_a(s):
        pltpu.make_async_copy(srcr_hbm.at[pl.ds(0, CH)], isrc[s], semA[s]).wait()
        pltpu.make_async_copy(sadj_hbm.at[pl.ds(0, CH)], iga[s], semA[s]).wait()
        pltpu.make_async_copy(dadj_hbm.at[pl.ds(0, CH)], igb[s], semA[s]).wait()
        pltpu.make_async_copy(elem_hbm.at[pl.ds(0, CH)],
                              elv[s].at[pl.ds(0, CH)], semA[s]).wait()

    def _issue_b(s):
        pltpu.async_copy(S_hbm.at[iga[s]], g1v[s], semB[s])
        pltpu.async_copy(T_hbm.at[igb[s]], g2v[s], semB[s])

    def _wait_b(s):
        pltpu.make_async_copy(S_hbm.at[pl.ds(0, CH)], g1v[s], semB[s]).wait()
        pltpu.make_async_copy(T_hbm.at[pl.ds(0, CH)], g2v[s], semB[s]).wait()

    def _issue_s(s):
        pltpu.async_copy(g2v[s], acc.at[isrc[s]], semS[s], add=True)

    def _wait_s(s):
        pltpu.make_async_copy(g2v[s], acc.at[pl.ds(0, CH)], semS[s]).wait()

    lane = lax.iota(jnp.int32, 16)

    def _compute(s):
        def _group(g, _):
            g16 = g * 16
            el = elv[s][pl.ds(g16, 16)]
            row = lane + g16
            sc = a2rep
            for jj in range(AH):
                cj = jnp.full((16,), jj, jnp.int32)
                u = (plsc.load_gather(g1v[s], [row, cj])
                     + plsc.load_gather(g2v[s], [row, cj + D])
                     + el * c_vec[jj])
                sc = sc + jnp.maximum(u, 0.0) * a2v[jj]
            ev = jnp.exp(jnp.where(sc > 0, sc, 0.2 * sc))
            scv[pl.ds(g16, 16)] = ev
            return 0

        lax.fori_loop(0, CH // 16, _group, 0)

        def _scale(i, _):
            e = scv[pl.ds(i, 16)][0]
            for k in range(D // 16):
                sl = pl.ds(k * 16, 16)
                g2v[s][i, sl] = g2v[s][i, sl] * e
            g2v[s][i, pl.ds(D, 16)] = jnp.where(lane == 0, e, 0.0)
            return 0

        lax.fori_loop(0, CH, _scale, 0)

    _issue_a(0, 0)
    _issue_a(1, 1)
    _wait_a(0)
    _issue_b(0)

    def _iter(j, s, s1, s2):
        @pl.when(_valid(j))
        def _():
            _wait_b(s)

        @pl.when(_valid(j + 1))
        def _():
            _wait_a(s1)
            _issue_b(s1)

        @pl.when(_valid(j))
        def _():
            _compute(s)
            _issue_s(s)

        @pl.when((j >= 1) & _valid(j - 1))
        def _():
            _wait_s(s2)

        @pl.when(_valid(j + 2))
        def _():
            _issue_a(j + 2, s2)

    def _outer(jj, _):
        j0 = jj * NSLOT
        for b in range(NSLOT):
            _iter(j0 + b, b % NSLOT, (b + 1) % NSLOT, (b + 2) % NSLOT)
        return 0

    lax.fori_loop(0, NOUTER, _outer, 0)
    plsc.subcore_barrier()

    def _dump(j, _):
        z = j * 16 + tid

        @pl.when(z < NZC)
        def _():
            pltpu.sync_copy(acc.at[pl.ds(z * CH, CH)], g2v[0])
            pltpu.sync_copy(g2v[0], out_hbm.at[pl.ds(hN + z * CH, CH)])

        if ZTAIL:
            @pl.when(z == NZC)
            def _():
                pltpu.sync_copy(acc.at[pl.ds(NZC * CH, ZTAIL)],
                                g2v[0].at[pl.ds(0, ZTAIL)])
                pltpu.sync_copy(g2v[0].at[pl.ds(0, ZTAIL)],
                                out_hbm.at[pl.ds(hN + NZC * CH, ZTAIL)])

        return 0

    lax.fori_loop(0, (NZC + 16) // 16, _dump, 0)


def _edge_pass(S2, T2, srcr, sadj2, dadj2, elem, consts):
    mesh = plsc.VectorSubcoreMesh(core_axis_name="c", subcore_axis_name="s")
    nsl = NSLOT
    k = pl.kernel(
        _edge_body,
        out_type=jax.ShapeDtypeStruct((H * N, R), jnp.float32),
        mesh=mesh,
        compiler_params=pltpu.CompilerParams(
            needs_layout_passes=False, use_tc_tiling_on_sc=False),
        scratch_types=[
            pltpu.VMEM_SHARED((N, R), jnp.float32),
            pltpu.VMEM((4, 16), jnp.float32),
            tuple(pltpu.VMEM((CH,), jnp.int32) for _ in range(nsl)),
            tuple(pltpu.VMEM((CH,), jnp.int32) for _ in range(nsl)),
            tuple(pltpu.VMEM((CH,), jnp.int32) for _ in range(nsl)),
            tuple(pltpu.VMEM((CH + 16,), jnp.float32) for _ in range(nsl)),
            tuple(pltpu.VMEM((CH, AH), jnp.float32) for _ in range(nsl)),
            tuple(pltpu.VMEM((CH, R), jnp.float32) for _ in range(nsl)),
            pltpu.VMEM((CH + 16,), jnp.float32),
            tuple(pltpu.SemaphoreType.DMA for _ in range(nsl)),
            tuple(pltpu.SemaphoreType.DMA for _ in range(nsl)),
            tuple(pltpu.SemaphoreType.DMA for _ in range(nsl)),
        ],
    )
    return k(S2, T2, srcr, sadj2, dadj2, elem, consts)


def _fin_body(P_ref, o_ref):
    pb = P_ref[0]
    o_ref[...] = pb[:, :D] / (pb[:, D:D + 1] + 1e-10)


def _finalize(P):
    grid = (H, N // BN)
    return pl.pallas_call(
        _fin_body,
        grid=grid,
        in_specs=[pl.BlockSpec((1, BN, R), lambda h, i: (h, i, 0))],
        out_specs=pl.BlockSpec((BN, D), lambda h, i: (i, h)),
        out_shape=jax.ShapeDtypeStruct((N, H * D), jnp.float32),
    )(P)


def kernel(x, idx, elem, W1, b1, W2, b2, A1, a1, A2, a2):
    A1s = A1[:, :D, :]
    A1d = A1[:, D:2 * D, :]
    c = A1[:, 2 * D, :]
    a2v = A2[:, :, 0]
    a2rep = jnp.broadcast_to(a2, (H, 16))
    consts = jnp.stack([c, a2v, a2rep, jnp.zeros_like(c)], axis=1)

    S, T = _prep(x, W1, b1, W2, b2, A1s, A1d, a1)
    S2 = S.reshape(H * N, AH)
    T2 = T.reshape(H * N, R)
    src = idx[0]
    dst = idx[1]
    sadj2 = jnp.concatenate([src, src + N]).astype(jnp.int32)
    dadj2 = jnp.concatenate([dst, dst + N]).astype(jnp.int32)
    P = _edge_pass(S2, T2, src, sadj2, dadj2, elem, consts)
    return _finalize(P.reshape(H, N, R))
